# baseline (speedup 1.0000x reference)
import functools

import jax
import jax.numpy as jnp
from jax import lax
from jax.experimental import pallas as pl
from jax.experimental.pallas import tpu as pltpu
from jax.experimental.pallas import tpu_sc as plsc

EMBEDDING_DIM = 256
CODEBOOK_SIZE = 8192
COMMITMENT_COST = 1.0

_BLOCK_N = 128
_WINDOW_K = 2048


def _vq_body(n_steps, xbf_ref, xs_ref, emb_ref, embbf_ref, enc_ref, idx_ref,
             loss_ref, es_ref):
    pid = pl.program_id(0)

    @pl.when(pid == 0)
    def _():
        emb = emb_ref[...]
        es_ref[...] = jnp.sum(emb * emb, axis=0, keepdims=True)

    sim = jnp.dot(xbf_ref[...], embbf_ref[...],
                  preferred_element_type=jnp.float32)
    x_sum = xs_ref[0, 0, :].reshape(-1, 1)
    d = (x_sum - 2.0 * sim) + es_ref[...]

    k = d.shape[1]
    bounds = list(range(0, k, _WINDOW_K))
    wmins = [jnp.min(d[:, lo:lo + _WINDOW_K], axis=1) for lo in bounds]
    acc = wmins[0].astype(jnp.bfloat16)
    val = wmins[0]
    wid = jnp.zeros_like(val, dtype=jnp.int32)
    for w in range(1, len(bounds)):
        take = wmins[w] < acc.astype(jnp.float32)
        val = jnp.where(take, wmins[w], val)
        wid = jnp.where(take, w, wid)
        acc = jnp.where(take, wmins[w].astype(jnp.bfloat16), acc)

    iota = jax.lax.broadcasted_iota(jnp.int32, d.shape, 1)
    hit = d == val[:, None]
    cand = jnp.where(hit, iota, k)
    idx = jnp.min(cand[:, 0:_WINDOW_K], axis=1)
    for w in range(1, len(bounds)):
        lo = bounds[w]
        cw = jnp.min(cand[:, lo:lo + _WINDOW_K], axis=1)
        idx = jnp.where(wid == w, cw, idx)
    idx_ref[...] = idx.reshape(1, 1, -1)

    enc_ref[...] = (iota == idx[:, None]).astype(jnp.float32)

    part = jnp.sum(val)
    prev = jnp.where(pid == 0, 0.0, loss_ref[0, 0])
    out = prev + part
    scale = COMMITMENT_COST / (n_steps * _BLOCK_N * EMBEDDING_DIM)
    out = jnp.where(pid == n_steps - 1, out * scale, out)
    loss_ref[...] = out.reshape(1, 1)


def _tc_call(flat_bf, xs3, embeddings, emb_bf):
    n, d = flat_bf.shape
    k = embeddings.shape[1]
    n_steps = n // _BLOCK_N
    return pl.pallas_call(
        functools.partial(_vq_body, n_steps),
        grid=(n_steps,),
        in_specs=[
            pl.BlockSpec((_BLOCK_N, d), lambda i: (i, 0)),
            pl.BlockSpec((1, 1, _BLOCK_N), lambda i: (i, 0, 0)),
            pl.BlockSpec((d, k), lambda i: (0, 0)),
            pl.BlockSpec((d, k), lambda i: (0, 0)),
        ],
        out_specs=[
            pl.BlockSpec((_BLOCK_N, k), lambda i: (i, 0)),
            pl.BlockSpec((1, 1, _BLOCK_N), lambda i: (i, 0, 0)),
            pl.BlockSpec((1, 1), lambda i: (0, 0)),
        ],
        out_shape=[
            jax.ShapeDtypeStruct((n, k), jnp.float32),
            jax.ShapeDtypeStruct((n_steps, 1, _BLOCK_N), jnp.int32),
            jax.ShapeDtypeStruct((1, 1), jnp.float32),
        ],
        scratch_shapes=[pltpu.VMEM((1, k), jnp.float32)],
    )(flat_bf, xs3, embeddings, emb_bf)


def _sc_gather(table, idx):
    v, d = table.shape
    b = idx.shape[0]
    info = plsc.get_sparse_core_info()
    nw = info.num_cores * info.num_subcores
    b_per_w = b // nw
    mesh = plsc.VectorSubcoreMesh(core_axis_name="c", subcore_axis_name="s")

    @functools.partial(
        pl.kernel,
        mesh=mesh,
        out_type=jax.ShapeDtypeStruct((b, d), jnp.float32),
        scratch_types=[
            pltpu.VMEM((b_per_w,), jnp.int32),
            pltpu.VMEM((b_per_w, d), jnp.float32),
            pltpu.SemaphoreType.DMA,
        ],
    )
    def gather_kernel(table_hbm, idx_hbm, out_hbm, idx_v, rows_v, sem):
        wid = lax.axis_index("s") * info.num_cores + lax.axis_index("c")
        base = wid * b_per_w
        pltpu.sync_copy(idx_hbm.at[pl.ds(base, b_per_w)], idx_v)
        pltpu.async_copy(table_hbm.at[idx_v], rows_v, sem).wait()
        pltpu.sync_copy(rows_v, out_hbm.at[pl.ds(base, b_per_w)])

    return gather_kernel(table, idx)


def kernel(inputs, embeddings):
    b, hw, d = inputs.shape
    n = b * hw
    flat = inputs.reshape(n, d)
    flat_bf = flat.astype(jnp.bfloat16)
    xs3 = jnp.sum(flat * flat, axis=1).reshape(n // _BLOCK_N, 1, _BLOCK_N)
    emb_bf = embeddings.astype(jnp.bfloat16)
    embt = embeddings.T

    enc, idx3, loss = _tc_call(flat_bf, xs3, embeddings, emb_bf)
    idx_flat = idx3.reshape(n)
    quant = _sc_gather(embt, idx_flat)

    quantized = quant.reshape(b, hw, d)
    indices = idx3.reshape(b, hw)
    return quantized, enc, indices, loss[0, 0]


# speedup vs baseline: 1.1092x; 1.1092x over previous
import functools

import jax
import jax.numpy as jnp
from jax import lax
from jax.experimental import pallas as pl
from jax.experimental.pallas import tpu as pltpu
from jax.experimental.pallas import tpu_sc as plsc

EMBEDDING_DIM = 256
CODEBOOK_SIZE = 8192
COMMITMENT_COST = 1.0

_BLOCK_N = 512
_WINDOW_K = 2048


def _vq_body(n_steps, xbf_ref, xs_ref, es_ref, embbf_ref, enc_ref, idx_ref,
             loss_ref):
    pid = pl.program_id(0)
    sim = jnp.dot(xbf_ref[...], embbf_ref[...],
                  preferred_element_type=jnp.float32)
    x_sum = xs_ref[0, 0, :].reshape(-1, 1)
    d = (x_sum - 2.0 * sim) + es_ref[...]

    k = d.shape[1]
    bounds = list(range(0, k, _WINDOW_K))
    wmins = [jnp.min(d[:, lo:lo + _WINDOW_K], axis=1) for lo in bounds]
    acc = wmins[0].astype(jnp.bfloat16)
    val = wmins[0]
    wid = jnp.zeros_like(val, dtype=jnp.int32)
    for w in range(1, len(bounds)):
        take = wmins[w] < acc.astype(jnp.float32)
        val = jnp.where(take, wmins[w], val)
        wid = jnp.where(take, w, wid)
        acc = jnp.where(take, wmins[w].astype(jnp.bfloat16), acc)

    iota = jax.lax.broadcasted_iota(jnp.int32, d.shape, 1)
    hit = d == val[:, None]
    cand = jnp.where(hit, iota, k)
    idx = jnp.min(cand[:, 0:_WINDOW_K], axis=1)
    for w in range(1, len(bounds)):
        lo = bounds[w]
        cw = jnp.min(cand[:, lo:lo + _WINDOW_K], axis=1)
        idx = jnp.where(wid == w, cw, idx)
    idx_ref[...] = idx.reshape(1, 1, -1)

    enc_ref[...] = (iota == idx[:, None]).astype(jnp.float32)

    part = jnp.sum(val)
    prev = jnp.where(pid == 0, 0.0, loss_ref[0, 0])
    out = prev + part
    scale = COMMITMENT_COST / (n_steps * _BLOCK_N * EMBEDDING_DIM)
    out = jnp.where(pid == n_steps - 1, out * scale, out)
    loss_ref[...] = out.reshape(1, 1)


def _tc_call(flat_bf, xs3, es2, emb_bf):
    n, d = flat_bf.shape
    k = emb_bf.shape[1]
    n_steps = n // _BLOCK_N
    return pl.pallas_call(
        functools.partial(_vq_body, n_steps),
        grid=(n_steps,),
        in_specs=[
            pl.BlockSpec((_BLOCK_N, d), lambda i: (i, 0)),
            pl.BlockSpec((1, 1, _BLOCK_N), lambda i: (i, 0, 0)),
            pl.BlockSpec((1, k), lambda i: (0, 0)),
            pl.BlockSpec((d, k), lambda i: (0, 0)),
        ],
        out_specs=[
            pl.BlockSpec((_BLOCK_N, k), lambda i: (i, 0)),
            pl.BlockSpec((1, 1, _BLOCK_N), lambda i: (i, 0, 0)),
            pl.BlockSpec((1, 1), lambda i: (0, 0)),
        ],
        out_shape=[
            jax.ShapeDtypeStruct((n, k), jnp.float32),
            jax.ShapeDtypeStruct((n_steps, 1, _BLOCK_N), jnp.int32),
            jax.ShapeDtypeStruct((1, 1), jnp.float32),
        ],
    )(flat_bf, xs3, es2, emb_bf)


def _sc_gather(table, idx):
    v, d = table.shape
    b = idx.shape[0]
    info = plsc.get_sparse_core_info()
    nw = info.num_cores * info.num_subcores
    b_per_w = b // nw
    mesh = plsc.VectorSubcoreMesh(core_axis_name="c", subcore_axis_name="s")

    @functools.partial(
        pl.kernel,
        mesh=mesh,
        out_type=jax.ShapeDtypeStruct((b, d), jnp.float32),
        scratch_types=[
            pltpu.VMEM((b_per_w,), jnp.int32),
            pltpu.VMEM((b_per_w, d), jnp.float32),
            pltpu.SemaphoreType.DMA,
        ],
    )
    def gather_kernel(table_hbm, idx_hbm, out_hbm, idx_v, rows_v, sem):
        wid = lax.axis_index("s") * info.num_cores + lax.axis_index("c")
        base = wid * b_per_w
        pltpu.sync_copy(idx_hbm.at[pl.ds(base, b_per_w)], idx_v)
        pltpu.async_copy(table_hbm.at[idx_v], rows_v, sem).wait()
        pltpu.sync_copy(rows_v, out_hbm.at[pl.ds(base, b_per_w)])

    return gather_kernel(table, idx)


def kernel(inputs, embeddings):
    b, hw, d = inputs.shape
    n = b * hw
    flat = inputs.reshape(n, d)
    flat_bf = flat.astype(jnp.bfloat16)
    xs3 = jnp.sum(flat * flat, axis=1).reshape(n // _BLOCK_N, 1, _BLOCK_N)
    es2 = jnp.sum(embeddings * embeddings, axis=0, keepdims=True)
    emb_bf = embeddings.astype(jnp.bfloat16)
    embt = embeddings.T

    enc, idx3, loss = _tc_call(flat_bf, xs3, es2, emb_bf)
    idx_flat = idx3.reshape(n)
    quant = _sc_gather(embt, idx_flat)

    quantized = quant.reshape(b, hw, d)
    indices = idx3.reshape(b, hw)
    return quantized, enc, indices, loss[0, 0]
